# Initial kernel scaffold; baseline (speedup 1.0000x reference)
import jax
import jax.numpy as jnp
from jax.experimental import pallas as pl


def kernel(node_embeddings, src_rids):
    raise NotImplementedError("write your pallas kernel here")


# speedup vs baseline: 5.7234x; 5.7234x over previous
import functools

import jax
import jax.numpy as jnp
from jax import lax
from jax.experimental import pallas as pl
from jax.experimental.pallas import tpu as pltpu
from jax.experimental.pallas import tpu_sc as plsc

VOCAB = 100000
HID = 128
SEQ = 200
BATCH = 1024

NUM_CORES = 2
NUM_SUBCORES = 16
NW = NUM_CORES * NUM_SUBCORES
NROWS = SEQ * BATCH
ROWS_PER_W = NROWS // NW
GROUP = 128
GROUPS_PER_W = ROWS_PER_W // GROUP

_mesh = plsc.VectorSubcoreMesh(
    core_axis_name="c", subcore_axis_name="s",
    num_cores=NUM_CORES, num_subcores=NUM_SUBCORES,
)


@functools.partial(
    pl.kernel,
    out_type=jax.ShapeDtypeStruct((NROWS, HID), jnp.float32),
    mesh=_mesh,
    scratch_types=[
        pltpu.VMEM((GROUPS_PER_W, GROUP), jnp.int32),
        pltpu.VMEM((GROUP, HID), jnp.float32),
        pltpu.SemaphoreType.DMA,
    ],
)
def _gather_kernel(table_hbm, idx_hbm, out_hbm, idx_v, rows_v, sem):
    wid = lax.axis_index("s") * NUM_CORES + lax.axis_index("c")
    grp_base = wid * GROUPS_PER_W
    pltpu.sync_copy(idx_hbm.at[wid], idx_v)

    def body(g, carry):
        row_off = (grp_base + g) * GROUP
        pltpu.async_copy(table_hbm.at[idx_v.at[g]], rows_v, sem).wait()
        pltpu.sync_copy(rows_v, out_hbm.at[pl.ds(row_off, GROUP)])
        return carry

    lax.fori_loop(0, GROUPS_PER_W, body, 0)


def kernel(node_embeddings, src_rids):
    idx = jnp.transpose(src_rids, (1, 0, 2)).reshape(NW, GROUPS_PER_W, GROUP)
    out = _gather_kernel(node_embeddings, idx)
    return out.reshape(BATCH, SEQ, HID)


# speedup vs baseline: 7.8995x; 1.3802x over previous
import functools

import jax
import jax.numpy as jnp
from jax import lax
from jax.experimental import pallas as pl
from jax.experimental.pallas import tpu as pltpu
from jax.experimental.pallas import tpu_sc as plsc

VOCAB = 100000
HID = 128
SEQ = 200
BATCH = 1024

NUM_CORES = 2
NUM_SUBCORES = 16
NW = NUM_CORES * NUM_SUBCORES
NROWS = SEQ * BATCH
ROWS_PER_W = NROWS // NW
GROUP = 128
GROUPS_PER_W = ROWS_PER_W // GROUP
NBUF = 5
NROUNDS = GROUPS_PER_W // NBUF

_mesh = plsc.VectorSubcoreMesh(
    core_axis_name="c", subcore_axis_name="s",
    num_cores=NUM_CORES, num_subcores=NUM_SUBCORES,
)


@functools.partial(
    pl.kernel,
    out_type=jax.ShapeDtypeStruct((NROWS, HID), jnp.float32),
    mesh=_mesh,
    scratch_types=[
        pltpu.VMEM((GROUPS_PER_W, GROUP), jnp.int32),
        pltpu.VMEM((NBUF, GROUP, HID), jnp.float32),
        pltpu.SemaphoreType.DMA((NBUF,)),
        pltpu.SemaphoreType.DMA((NBUF,)),
    ],
)
def _gather_kernel(table_hbm, idx_hbm, out_hbm, idx_v, rows_v, gsem, ssem):
    wid = lax.axis_index("s") * NUM_CORES + lax.axis_index("c")
    grp_base = wid * GROUPS_PER_W
    pltpu.sync_copy(idx_hbm.at[wid], idx_v)

    for b in range(NBUF):
        pltpu.async_copy(table_hbm.at[idx_v.at[b]], rows_v.at[b], gsem.at[b])

    def body(i, carry):
        for b in range(NBUF):
            g = i * NBUF + b
            row_off = (grp_base + g) * GROUP
            out_slice = out_hbm.at[pl.ds(row_off, GROUP)]
            pltpu.make_async_copy(
                table_hbm.at[idx_v.at[b]], rows_v.at[b], gsem.at[b]
            ).wait()
            pltpu.async_copy(rows_v.at[b], out_slice, ssem.at[b])
            pltpu.make_async_copy(rows_v.at[b], out_slice, ssem.at[b]).wait()

            @pl.when(g + NBUF < GROUPS_PER_W)
            def _():
                pltpu.async_copy(
                    table_hbm.at[idx_v.at[g + NBUF]],
                    rows_v.at[b],
                    gsem.at[b],
                )

        return carry

    lax.fori_loop(0, NROUNDS, body, 0)


def kernel(node_embeddings, src_rids):
    idx = jnp.transpose(src_rids, (1, 0, 2)).reshape(NW, GROUPS_PER_W, GROUP)
    out = _gather_kernel(node_embeddings, idx)
    return out.reshape(BATCH, SEQ, HID)
